# baseline (speedup 1.0000x reference)
import functools

import jax
import jax.numpy as jnp
from jax.experimental import pallas as pl
from jax.experimental.pallas import tpu as pltpu
from jax.experimental.pallas import tpu_sc as plsc

_B, _D_IN, _D_HID, _L, _K = 4096, 2048, 1024, 64, 512
_BLK = 512
_NB = _B // _BLK


def _onehot(ind_ref):
    ind = ind_ref[0, 0, :]
    iota = jax.lax.broadcasted_iota(jnp.int32, (_BLK, _K), 1)
    return (iota == ind[:, None]).astype(jnp.float32)


def _stage_a(x_ref, enc_w_ref, enc_b_ref, fc1_w_ref, fc1_b_ref,
             fc2_w_ref, fc2_b_ref, ctx_ref,
             feat_ref, bbn_ref, cbn_ref, e_ref, ind_ref, s_ref, wpc_ref,
             cnt_acc):
    i = pl.program_id(0)
    x = x_ref[...]
    feat = jnp.maximum(
        jnp.dot(x, enc_w_ref[...], preferred_element_type=jnp.float32)
        + enc_b_ref[...], 0.0)
    feat_ref[...] = feat
    bbn = (jnp.dot(feat, fc1_w_ref[...], preferred_element_type=jnp.float32)
           + fc1_b_ref[...])
    cbn = (jnp.dot(feat, fc2_w_ref[...], preferred_element_type=jnp.float32)
           + fc2_b_ref[...])
    bbn_ref[...] = bbn
    cbn_ref[...] = cbn
    ctx = ctx_ref[...]
    cross = jax.lax.dot_general(bbn, ctx, (((1,), (1,)), ((), ())),
                                preferred_element_type=jnp.float32)
    d2 = (jnp.sum(bbn * bbn, axis=1, keepdims=True)
          - 2.0 * cross
          + jnp.sum(ctx * ctx, axis=1)[None, :])
    m = jnp.min(d2, axis=1, keepdims=True)
    iota = jax.lax.broadcasted_iota(jnp.int32, (_BLK, _K), 1)
    ind = jnp.min(jnp.where(d2 <= m, iota, _K), axis=1)
    e = (iota == ind[:, None]).astype(jnp.float32)
    e_ref[...] = e
    ind_ref[...] = ind[None, None, :]

    @pl.when(i == 0)
    def _():
        cnt_acc[...] = jnp.zeros_like(cnt_acc)

    cnt_acc[...] += jnp.sum(e, axis=0, keepdims=True)

    @pl.when(i == _NB - 1)
    def _():
        m1 = ctx - 1.0
        a = (jax.lax.dot_general(ctx, ctx, (((1,), (1,)), ((), ())),
                                 preferred_element_type=jnp.float32)
             + jax.lax.dot_general(m1, m1, (((1,), (1,)), ((), ())),
                                   preferred_element_type=jnp.float32)
             ) / jnp.float32(_L)
        s = jnp.exp(1.4 * jnp.log(jnp.maximum(a, 1e-12)))
        s_ref[...] = s
        degc = jnp.sum(s * cnt_acc[...], axis=1, keepdims=True)
        wpc_ref[...] = jax.lax.rsqrt(degc)


def _stage_bgt(ind_ref, s_ref, gt_ref):
    e = _onehot(ind_ref)
    gt_ref[...] = jax.lax.dot_general(s_ref[...], e, (((1,), (1,)), ((), ())),
                                      preferred_element_type=jnp.float32)


def _stage_b(ind_ref, cbn_ref, wpc_ref, s_ref, gcn_w_ref,
             w_ref, p_ref, t_acc):
    i = pl.program_id(0)
    e = _onehot(ind_ref)
    w = jnp.dot(e, wpc_ref[...], preferred_element_type=jnp.float32)
    w_ref[...] = w
    wc = w * cbn_ref[...]

    @pl.when(i == 0)
    def _():
        t_acc[...] = jnp.zeros_like(t_acc)

    t_acc[...] += jax.lax.dot_general(e, wc, (((0,), (0,)), ((), ())),
                                      preferred_element_type=jnp.float32)

    @pl.when(i == _NB - 1)
    def _():
        u = jnp.dot(s_ref[...], t_acc[...],
                    preferred_element_type=jnp.float32)
        p_ref[...] = jnp.dot(u, gcn_w_ref[...],
                             preferred_element_type=jnp.float32)


def _stage_c(ind_ref, w_ref, p_ref, dec_w_ref, dec_b_ref, dec_ref):
    e = _onehot(ind_ref)
    r = jnp.dot(e, p_ref[...], preferred_element_type=jnp.float32)
    latent = jax.nn.sigmoid(w_ref[...] * r)
    dec_ref[...] = (jnp.dot(latent, dec_w_ref[...],
                            preferred_element_type=jnp.float32)
                    + dec_b_ref[...])


_NC, _NS = 2, 16
_NW = _NC * _NS
_RPW = _B // _NW
_CH = 8
_NCH = _RPW // _CH


_NBUF = 3


def _sc_adj(gt_hbm, ind_hbm, adj_hbm, idx_v, buf0, buf1, buf2,
            gsem0, gsem1, gsem2, ssem0, ssem1, ssem2):
    wid = jax.lax.axis_index("s") * _NC + jax.lax.axis_index("c")
    base = wid * _RPW
    pltpu.sync_copy(ind_hbm.at[pl.ds(base, _RPW)], idx_v)
    bufs = (buf0, buf1, buf2)
    gsems = (gsem0, gsem1, gsem2)
    ssems = (ssem0, ssem1, ssem2)
    gath = [None] * _NCH
    scat = [None] * _NCH

    def _gather(c):
        return pltpu.async_copy(
            gt_hbm.at[idx_v.at[pl.ds(c * _CH, _CH)]],
            bufs[c % _NBUF], gsems[c % _NBUF])

    for c in range(_NBUF - 1):
        gath[c] = _gather(c)
    for c in range(_NCH):
        nxt = c + _NBUF - 1
        if nxt < _NCH:
            if nxt >= _NBUF:
                scat[nxt - _NBUF].wait()
            gath[nxt] = _gather(nxt)
        gath[c].wait()
        scat[c] = pltpu.async_copy(
            bufs[c % _NBUF], adj_hbm.at[pl.ds(base + c * _CH, _CH)],
            ssems[c % _NBUF])
    for c in range(_NCH - min(_NBUF, _NCH), _NCH):
        scat[c].wait()


def kernel(inputs, enc_w, enc_b, fc1_w, fc1_b, fc2_w, fc2_b,
           gcn_w, dec_w, dec_b, context):
    f32 = jnp.float32
    enc_b2 = enc_b.reshape(1, _D_HID)
    fc1_b2 = fc1_b.reshape(1, _L)
    fc2_b2 = fc2_b.reshape(1, _L)
    dec_b2 = dec_b.reshape(1, _D_IN)

    whole = lambda *shape: pl.BlockSpec(shape, lambda i: (0,) * len(shape))
    ind_spec = pl.BlockSpec((1, 1, _BLK), lambda i: (i, 0, 0))

    feat, bbn, cbn, e, ind3, s, wpc = pl.pallas_call(
        _stage_a,
        grid=(_NB,),
        in_specs=[
            pl.BlockSpec((_BLK, _D_IN), lambda i: (i, 0)),
            whole(_D_IN, _D_HID),
            whole(1, _D_HID),
            whole(_D_HID, _L),
            whole(1, _L),
            whole(_D_HID, _L),
            whole(1, _L),
            whole(_K, _L),
        ],
        out_specs=[
            pl.BlockSpec((_BLK, _D_HID), lambda i: (i, 0)),
            pl.BlockSpec((_BLK, _L), lambda i: (i, 0)),
            pl.BlockSpec((_BLK, _L), lambda i: (i, 0)),
            pl.BlockSpec((_BLK, _K), lambda i: (i, 0)),
            ind_spec,
            whole(_K, _K),
            whole(_K, 1),
        ],
        out_shape=[
            jax.ShapeDtypeStruct((_B, _D_HID), f32),
            jax.ShapeDtypeStruct((_B, _L), f32),
            jax.ShapeDtypeStruct((_B, _L), f32),
            jax.ShapeDtypeStruct((_B, _K), f32),
            jax.ShapeDtypeStruct((_NB, 1, _BLK), jnp.int32),
            jax.ShapeDtypeStruct((_K, _K), f32),
            jax.ShapeDtypeStruct((_K, 1), f32),
        ],
        scratch_shapes=[pltpu.VMEM((1, _K), f32)],
    )(inputs, enc_w, enc_b2, fc1_w, fc1_b2, fc2_w, fc2_b2, context)
    ind = ind3.reshape(_B)

    gt = pl.pallas_call(
        _stage_bgt,
        grid=(_NB,),
        in_specs=[ind_spec, whole(_K, _K)],
        out_specs=[pl.BlockSpec((_K, _BLK), lambda i: (0, i))],
        out_shape=[jax.ShapeDtypeStruct((_K, _B), f32)],
    )(ind3, s)[0]

    adj = pl.kernel(
        _sc_adj,
        out_type=jax.ShapeDtypeStruct((_B, _B), f32),
        mesh=plsc.VectorSubcoreMesh(core_axis_name="c", subcore_axis_name="s"),
        scratch_types=[
            pltpu.VMEM((_RPW,), jnp.int32),
            pltpu.VMEM((_CH, _B), f32),
            pltpu.VMEM((_CH, _B), f32),
            pltpu.VMEM((_CH, _B), f32),
            pltpu.SemaphoreType.DMA,
            pltpu.SemaphoreType.DMA,
            pltpu.SemaphoreType.DMA,
            pltpu.SemaphoreType.DMA,
            pltpu.SemaphoreType.DMA,
            pltpu.SemaphoreType.DMA,
        ],
    )(gt, ind)

    w, p = pl.pallas_call(
        _stage_b,
        grid=(_NB,),
        in_specs=[
            ind_spec,
            pl.BlockSpec((_BLK, _L), lambda i: (i, 0)),
            whole(_K, 1),
            whole(_K, _K),
            whole(_L, _L),
        ],
        out_specs=[
            pl.BlockSpec((_BLK, 1), lambda i: (i, 0)),
            whole(_K, _L),
        ],
        out_shape=[
            jax.ShapeDtypeStruct((_B, 1), f32),
            jax.ShapeDtypeStruct((_K, _L), f32),
        ],
        scratch_shapes=[pltpu.VMEM((_K, _L), f32)],
    )(ind3, cbn, wpc, s, gcn_w)

    decoded = pl.pallas_call(
        _stage_c,
        grid=(_NB,),
        in_specs=[
            ind_spec,
            pl.BlockSpec((_BLK, 1), lambda i: (i, 0)),
            whole(_K, _L),
            whole(_L, _D_IN),
            whole(1, _D_IN),
        ],
        out_specs=[pl.BlockSpec((_BLK, _D_IN), lambda i: (i, 0))],
        out_shape=[jax.ShapeDtypeStruct((_B, _D_IN), f32)],
    )(ind3, w, p, dec_w, dec_b2)[0]

    return decoded, bbn, e, feat, adj


# speedup vs baseline: 1.0453x; 1.0453x over previous
import functools

import jax
import jax.numpy as jnp
from jax.experimental import pallas as pl
from jax.experimental.pallas import tpu as pltpu
from jax.experimental.pallas import tpu_sc as plsc

_B, _D_IN, _D_HID, _L, _K = 4096, 2048, 1024, 64, 512
_BLK = 512
_NB = _B // _BLK


def _onehot(ind_ref):
    ind = ind_ref[0, 0, :]
    iota = jax.lax.broadcasted_iota(jnp.int32, (_BLK, _K), 1)
    return (iota == ind[:, None]).astype(jnp.float32)


def _stage_s(ctx_ref, s_ref):
    ctx = ctx_ref[...]
    m1 = ctx - 1.0
    a = (jax.lax.dot_general(ctx, ctx, (((1,), (1,)), ((), ())),
                             preferred_element_type=jnp.float32)
         + jax.lax.dot_general(m1, m1, (((1,), (1,)), ((), ())),
                               preferred_element_type=jnp.float32)
         ) / jnp.float32(_L)
    s_ref[...] = jnp.exp(1.4 * jnp.log(jnp.maximum(a, 1e-12)))


def _stage_a(x_ref, enc_w_ref, enc_b_ref, fc1_w_ref, fc1_b_ref,
             fc2_w_ref, fc2_b_ref, ctx_ref, s_ref,
             feat_ref, bbn_ref, cbn_ref, e_ref, ind_ref, gt_ref, wpc_ref,
             cnt_acc):
    i = pl.program_id(0)
    x = x_ref[...]
    feat = jnp.maximum(
        jnp.dot(x, enc_w_ref[...], preferred_element_type=jnp.float32)
        + enc_b_ref[...], 0.0)
    feat_ref[...] = feat
    bbn = (jnp.dot(feat, fc1_w_ref[...], preferred_element_type=jnp.float32)
           + fc1_b_ref[...])
    cbn = (jnp.dot(feat, fc2_w_ref[...], preferred_element_type=jnp.float32)
           + fc2_b_ref[...])
    bbn_ref[...] = bbn
    cbn_ref[...] = cbn
    ctx = ctx_ref[...]
    cross = jax.lax.dot_general(bbn, ctx, (((1,), (1,)), ((), ())),
                                preferred_element_type=jnp.float32)
    d2 = (jnp.sum(bbn * bbn, axis=1, keepdims=True)
          - 2.0 * cross
          + jnp.sum(ctx * ctx, axis=1)[None, :])
    m = jnp.min(d2, axis=1, keepdims=True)
    iota = jax.lax.broadcasted_iota(jnp.int32, (_BLK, _K), 1)
    ind = jnp.min(jnp.where(d2 <= m, iota, _K), axis=1)
    e = (iota == ind[:, None]).astype(jnp.float32)
    e_ref[...] = e
    ind_ref[...] = ind[None, None, :]
    gt_ref[...] = jax.lax.dot_general(s_ref[...], e, (((1,), (1,)), ((), ())),
                                      preferred_element_type=jnp.float32)

    @pl.when(i == 0)
    def _():
        cnt_acc[...] = jnp.zeros_like(cnt_acc)

    cnt_acc[...] += jnp.sum(e, axis=0, keepdims=True)

    @pl.when(i == _NB - 1)
    def _():
        degc = jnp.sum(s_ref[...] * cnt_acc[...], axis=1, keepdims=True)
        wpc_ref[...] = jax.lax.rsqrt(degc)


def _stage_b(ind_ref, cbn_ref, wpc_ref, s_ref, gcn_w_ref,
             w_ref, p_ref, t_acc):
    i = pl.program_id(0)
    e = _onehot(ind_ref)
    w = jnp.dot(e, wpc_ref[...], preferred_element_type=jnp.float32)
    w_ref[...] = w
    wc = w * cbn_ref[...]

    @pl.when(i == 0)
    def _():
        t_acc[...] = jnp.zeros_like(t_acc)

    t_acc[...] += jax.lax.dot_general(e, wc, (((0,), (0,)), ((), ())),
                                      preferred_element_type=jnp.float32)

    @pl.when(i == _NB - 1)
    def _():
        u = jnp.dot(s_ref[...], t_acc[...],
                    preferred_element_type=jnp.float32)
        p_ref[...] = jnp.dot(u, gcn_w_ref[...],
                             preferred_element_type=jnp.float32)


def _stage_c(ind_ref, w_ref, p_ref, dec_w_ref, dec_b_ref, dec_ref):
    e = _onehot(ind_ref)
    r = jnp.dot(e, p_ref[...], preferred_element_type=jnp.float32)
    latent = jax.nn.sigmoid(w_ref[...] * r)
    dec_ref[...] = (jnp.dot(latent, dec_w_ref[...],
                            preferred_element_type=jnp.float32)
                    + dec_b_ref[...])


_NC, _NS = 2, 16
_NW = _NC * _NS
_RPW = _B // _NW
_CH = 8
_NCH = _RPW // _CH


_NBUF = 3


def _sc_adj(gt_hbm, ind_hbm, adj_hbm, idx_v, buf0, buf1, buf2,
            gsem0, gsem1, gsem2, ssem0, ssem1, ssem2):
    wid = jax.lax.axis_index("s") * _NC + jax.lax.axis_index("c")
    base = wid * _RPW
    pltpu.sync_copy(ind_hbm.at[pl.ds(base, _RPW)], idx_v)
    bufs = (buf0, buf1, buf2)
    gsems = (gsem0, gsem1, gsem2)
    ssems = (ssem0, ssem1, ssem2)
    gath = [None] * _NCH
    scat = [None] * _NCH

    def _gather(c):
        return pltpu.async_copy(
            gt_hbm.at[idx_v.at[pl.ds(c * _CH, _CH)]],
            bufs[c % _NBUF], gsems[c % _NBUF])

    for c in range(_NBUF - 1):
        gath[c] = _gather(c)
    for c in range(_NCH):
        nxt = c + _NBUF - 1
        if nxt < _NCH:
            if nxt >= _NBUF:
                scat[nxt - _NBUF].wait()
            gath[nxt] = _gather(nxt)
        gath[c].wait()
        scat[c] = pltpu.async_copy(
            bufs[c % _NBUF], adj_hbm.at[pl.ds(base + c * _CH, _CH)],
            ssems[c % _NBUF])
    for c in range(_NCH - min(_NBUF, _NCH), _NCH):
        scat[c].wait()


def kernel(inputs, enc_w, enc_b, fc1_w, fc1_b, fc2_w, fc2_b,
           gcn_w, dec_w, dec_b, context):
    f32 = jnp.float32
    enc_b2 = enc_b.reshape(1, _D_HID)
    fc1_b2 = fc1_b.reshape(1, _L)
    fc2_b2 = fc2_b.reshape(1, _L)
    dec_b2 = dec_b.reshape(1, _D_IN)

    whole = lambda *shape: pl.BlockSpec(shape, lambda i: (0,) * len(shape))
    ind_spec = pl.BlockSpec((1, 1, _BLK), lambda i: (i, 0, 0))

    s = pl.pallas_call(
        _stage_s,
        in_specs=[pl.BlockSpec((_K, _L), lambda: (0, 0))],
        out_specs=pl.BlockSpec((_K, _K), lambda: (0, 0)),
        out_shape=jax.ShapeDtypeStruct((_K, _K), f32),
    )(context)

    feat, bbn, cbn, e, ind3, gt, wpc = pl.pallas_call(
        _stage_a,
        grid=(_NB,),
        in_specs=[
            pl.BlockSpec((_BLK, _D_IN), lambda i: (i, 0)),
            whole(_D_IN, _D_HID),
            whole(1, _D_HID),
            whole(_D_HID, _L),
            whole(1, _L),
            whole(_D_HID, _L),
            whole(1, _L),
            whole(_K, _L),
            whole(_K, _K),
        ],
        out_specs=[
            pl.BlockSpec((_BLK, _D_HID), lambda i: (i, 0)),
            pl.BlockSpec((_BLK, _L), lambda i: (i, 0)),
            pl.BlockSpec((_BLK, _L), lambda i: (i, 0)),
            pl.BlockSpec((_BLK, _K), lambda i: (i, 0)),
            ind_spec,
            pl.BlockSpec((_K, _BLK), lambda i: (0, i)),
            whole(_K, 1),
        ],
        out_shape=[
            jax.ShapeDtypeStruct((_B, _D_HID), f32),
            jax.ShapeDtypeStruct((_B, _L), f32),
            jax.ShapeDtypeStruct((_B, _L), f32),
            jax.ShapeDtypeStruct((_B, _K), f32),
            jax.ShapeDtypeStruct((_NB, 1, _BLK), jnp.int32),
            jax.ShapeDtypeStruct((_K, _B), f32),
            jax.ShapeDtypeStruct((_K, 1), f32),
        ],
        scratch_shapes=[pltpu.VMEM((1, _K), f32)],
    )(inputs, enc_w, enc_b2, fc1_w, fc1_b2, fc2_w, fc2_b2, context, s)
    ind = ind3.reshape(_B)

    adj = pl.kernel(
        _sc_adj,
        out_type=jax.ShapeDtypeStruct((_B, _B), f32),
        mesh=plsc.VectorSubcoreMesh(core_axis_name="c", subcore_axis_name="s"),
        scratch_types=[
            pltpu.VMEM((_RPW,), jnp.int32),
            pltpu.VMEM((_CH, _B), f32),
            pltpu.VMEM((_CH, _B), f32),
            pltpu.VMEM((_CH, _B), f32),
            pltpu.SemaphoreType.DMA,
            pltpu.SemaphoreType.DMA,
            pltpu.SemaphoreType.DMA,
            pltpu.SemaphoreType.DMA,
            pltpu.SemaphoreType.DMA,
            pltpu.SemaphoreType.DMA,
        ],
    )(gt, ind)

    w, p = pl.pallas_call(
        _stage_b,
        grid=(_NB,),
        in_specs=[
            ind_spec,
            pl.BlockSpec((_BLK, _L), lambda i: (i, 0)),
            whole(_K, 1),
            whole(_K, _K),
            whole(_L, _L),
        ],
        out_specs=[
            pl.BlockSpec((_BLK, 1), lambda i: (i, 0)),
            whole(_K, _L),
        ],
        out_shape=[
            jax.ShapeDtypeStruct((_B, 1), f32),
            jax.ShapeDtypeStruct((_K, _L), f32),
        ],
        scratch_shapes=[pltpu.VMEM((_K, _L), f32)],
    )(ind3, cbn, wpc, s, gcn_w)

    decoded = pl.pallas_call(
        _stage_c,
        grid=(_NB,),
        in_specs=[
            ind_spec,
            pl.BlockSpec((_BLK, 1), lambda i: (i, 0)),
            whole(_K, _L),
            whole(_L, _D_IN),
            whole(1, _D_IN),
        ],
        out_specs=[pl.BlockSpec((_BLK, _D_IN), lambda i: (i, 0))],
        out_shape=[jax.ShapeDtypeStruct((_B, _D_IN), f32)],
    )(ind3, w, p, dec_w, dec_b2)[0]

    return decoded, bbn, e, feat, adj
